# Initial kernel scaffold; baseline (speedup 1.0000x reference)
import jax
import jax.numpy as jnp
from jax.experimental import pallas as pl


def kernel(x, edge_index, W1, b1, Wc, bc):
    raise NotImplementedError("write your pallas kernel here")


# speedup vs baseline: 15.1192x; 15.1192x over previous
import functools

import jax
import jax.numpy as jnp
from jax import lax
from jax.experimental import pallas as pl
from jax.experimental.pallas import tpu as pltpu
from jax.experimental.pallas import tpu_sc as plsc

_NC = 2
_NS = 16
_NW = _NC * _NS

_K = 80


def _mesh():
    return plsc.VectorSubcoreMesh(core_axis_name="c", subcore_axis_name="s")


def _make_hist(n, e, d):
    ew = e // _NW
    nch = ew // _K
    rpt = n // _NS

    @functools.partial(
        pl.kernel,
        out_type=jax.ShapeDtypeStruct((_NC, _NS, rpt, d), jnp.float32),
        mesh=_mesh(),
        scratch_types=[
            pltpu.VMEM((_K,), jnp.int32),
            pltpu.VMEM((_K, d), jnp.float32),
            pltpu.VMEM_SHARED((n, d), jnp.float32),
        ],
    )
    def hist(dst_hbm, zeros_hbm, ones_hbm, out_hbm, didx, ones, hist_sh):
        c = lax.axis_index("c")
        s = lax.axis_index("s")
        wid = s * _NC + c
        pltpu.sync_copy(zeros_hbm, hist_sh.at[pl.ds(s * rpt, rpt)])
        pltpu.sync_copy(ones_hbm, ones)
        plsc.subcore_barrier()
        base = wid * ew

        def body(i, carry):
            off = base + i * _K
            pltpu.sync_copy(dst_hbm.at[pl.ds(off, _K)], didx)
            pltpu.sync_copy(ones, hist_sh.at[didx], add=True)
            return carry

        lax.fori_loop(0, nch, body, 0)
        plsc.subcore_barrier()
        pltpu.sync_copy(hist_sh.at[pl.ds(s * rpt, rpt)], out_hbm.at[c, s])

    return hist


def _make_seg(n, e, d):
    ew = e // _NW
    nch = ew // _K
    rpt = n // _NS

    @functools.partial(
        pl.kernel,
        out_type=jax.ShapeDtypeStruct((_NC, _NS, rpt, d), jnp.float32),
        mesh=_mesh(),
        scratch_types=[
            pltpu.VMEM((_K,), jnp.int32),
            pltpu.VMEM((_K,), jnp.int32),
            pltpu.VMEM((_K, d), jnp.float32),
            pltpu.VMEM_SHARED((n, d), jnp.float32),
            pltpu.SemaphoreType.DMA,
        ],
    )
    def seg(g_hbm, src_hbm, dst_hbm, zeros_hbm, out_hbm,
            sidx, didx, rows, acc_sh, sem):
        c = lax.axis_index("c")
        s = lax.axis_index("s")
        wid = s * _NC + c
        pltpu.sync_copy(zeros_hbm, acc_sh.at[pl.ds(s * rpt, rpt)])
        plsc.subcore_barrier()
        base = wid * ew

        def body(i, carry):
            off = base + i * _K
            pltpu.sync_copy(src_hbm.at[pl.ds(off, _K)], sidx)
            pltpu.sync_copy(dst_hbm.at[pl.ds(off, _K)], didx)
            pltpu.async_copy(g_hbm.at[sidx], rows, sem).wait()
            pltpu.sync_copy(rows, acc_sh.at[didx], add=True)
            return carry

        lax.fori_loop(0, nch, body, 0)
        plsc.subcore_barrier()
        pltpu.sync_copy(acc_sh.at[pl.ds(s * rpt, rpt)], out_hbm.at[c, s])

    return seg


def _enc_body(deg_ref, x_ref, w1_ref, g_ref, dinv_ref):
    deg = deg_ref[0, :, 0] + deg_ref[1, :, 0] + 1.0
    dinv = lax.rsqrt(deg)
    g_ref[...] = jnp.dot(x_ref[...] * dinv[:, None], w1_ref[...],
                         preferred_element_type=jnp.float32)
    dinv_ref[...] = dinv[:, None]


def _encode(deg_parts, x, w1, bn, bm):
    n, d = x.shape
    h = w1.shape[1]
    grid = (n // bm,)
    return pl.pallas_call(
        _enc_body,
        grid=grid,
        in_specs=[
            pl.BlockSpec((_NC, bm, bn), lambda i: (0, i, 0)),
            pl.BlockSpec((bm, d), lambda i: (i, 0)),
            pl.BlockSpec((d, h), lambda i: (0, 0)),
        ],
        out_specs=[
            pl.BlockSpec((bm, h), lambda i: (i, 0)),
            pl.BlockSpec((bm, 1), lambda i: (i, 0)),
        ],
        out_shape=[
            jax.ShapeDtypeStruct((n, h), jnp.float32),
            jax.ShapeDtypeStruct((n, 1), jnp.float32),
        ],
    )(deg_parts, x, w1)


def _dec_body(acc_ref, g_ref, dinv_ref, b1_ref, wc_ref, bc_ref, out_ref):
    tot = acc_ref[0] + acc_ref[1] + g_ref[...]
    pre = tot * dinv_ref[...] + b1_ref[...]
    out_ref[...] = jnp.dot(jnp.maximum(pre, 0.0), wc_ref[...],
                           preferred_element_type=jnp.float32) + bc_ref[...]


def _decode(acc_parts, g, dinv, b1, wc, bc, bm):
    n, h = g.shape
    o = wc.shape[1]
    grid = (n // bm,)
    return pl.pallas_call(
        _dec_body,
        grid=grid,
        in_specs=[
            pl.BlockSpec((_NC, bm, h), lambda i: (0, i, 0)),
            pl.BlockSpec((bm, h), lambda i: (i, 0)),
            pl.BlockSpec((bm, 1), lambda i: (i, 0)),
            pl.BlockSpec((1, h), lambda i: (0, 0)),
            pl.BlockSpec((h, o), lambda i: (0, 0)),
            pl.BlockSpec((1, o), lambda i: (0, 0)),
        ],
        out_specs=pl.BlockSpec((bm, o), lambda i: (i, 0)),
        out_shape=jax.ShapeDtypeStruct((n, o), jnp.float32),
    )(acc_parts, g, dinv, b1, wc, bc)


def kernel(x, edge_index, W1, b1, Wc, bc):
    n, d = x.shape
    h = W1.shape[1]
    e = edge_index.shape[1]
    assert e % (_NW * _K) == 0 and n % _NS == 0

    src = edge_index[0].astype(jnp.int32)
    dst = edge_index[1].astype(jnp.int32)

    rpt = n // _NS
    ones_d = jnp.ones((_K, d), jnp.float32)
    zeros_d = jnp.zeros((rpt, d), jnp.float32)

    deg_parts = _make_hist(n, e, d)(dst, zeros_d, ones_d).reshape(_NC, n, d)
    g, dinv = _encode(deg_parts, x, W1, d, 1000)
    acc_parts = _make_seg(n, e, h)(g, src, dst, zeros_d).reshape(_NC, n, h)
    out = _decode(acc_parts, g, dinv,
                  b1.reshape(1, h), Wc, bc.reshape(1, -1), 1000)
    return out


# speedup vs baseline: 15.4579x; 1.0224x over previous
import functools

import jax
import jax.numpy as jnp
from jax import lax
from jax.experimental import pallas as pl
from jax.experimental.pallas import tpu as pltpu
from jax.experimental.pallas import tpu_sc as plsc

_NC = 2
_NS = 16
_NW = _NC * _NS

_K = 128


def _mesh():
    return plsc.VectorSubcoreMesh(core_axis_name="c", subcore_axis_name="s")


def _pack_edges(src, dst, n):
    e = src.shape[0]
    ew = -(-e // _NW)
    ewp = -(-ew // (2 * _K)) * (2 * _K)
    packed = src | (dst << 16)
    fill = jnp.int32(n << 16)
    pad = jnp.full((_NW * ewp - e,), fill, jnp.int32)
    return jnp.concatenate([packed, pad]), ewp


def _make_hist(n, ewp, zr):
    ncw = ewp // _K

    @functools.partial(
        pl.kernel,
        out_type=jax.ShapeDtypeStruct((_NC * _NS * zr,), jnp.float32),
        mesh=_mesh(),
        scratch_types=[
            pltpu.VMEM((ewp,), jnp.int32),
            pltpu.VMEM((_K,), jnp.int32),
            pltpu.VMEM((_K,), jnp.float32),
            pltpu.VMEM((zr,), jnp.float32),
            pltpu.VMEM_SHARED((_NS * zr,), jnp.float32),
        ],
    )
    def hist(pk_hbm, out_hbm, pflat, didx_c, ones, zbuf, flat_sh):
        c = lax.axis_index("c")
        s = lax.axis_index("s")
        wid = s * _NC + c
        pltpu.sync_copy(pk_hbm.at[pl.ds(wid * ewp, ewp)], pflat)

        def fill1(j, carry):
            ones[pl.ds(j * 16, 16)] = jnp.full((16,), 1.0, jnp.float32)
            return carry

        lax.fori_loop(0, _K // 16, fill1, 0)

        def fill0(j, carry):
            zbuf[pl.ds(j * 16, 16)] = jnp.zeros((16,), jnp.float32)
            return carry

        lax.fori_loop(0, zr // 16, fill0, 0)
        pltpu.sync_copy(zbuf, flat_sh.at[pl.ds(s * zr, zr)])
        plsc.subcore_barrier()

        def body(i, carry):
            for t in range(_K // 16):
                v = pflat[pl.ds(i * _K + t * 16, 16)]
                didx_c[pl.ds(t * 16, 16)] = lax.shift_right_logical(v, 16)
            pltpu.sync_copy(ones, flat_sh.at[didx_c], add=True)
            return carry

        lax.fori_loop(0, ncw, body, 0)
        plsc.subcore_barrier()
        pltpu.sync_copy(flat_sh.at[pl.ds(s * zr, zr)],
                        out_hbm.at[pl.ds((c * _NS + s) * zr, zr)])

    return hist


def _make_seg(n, ewp, d):
    rpt = n // _NS
    zq = 5
    zrow = rpt // zq
    ncw = ewp // _K

    @functools.partial(
        pl.kernel,
        out_type=jax.ShapeDtypeStruct((_NC, _NS, rpt, d), jnp.float32),
        mesh=_mesh(),
        scratch_types=[
            pltpu.VMEM((ewp,), jnp.int32),
            pltpu.VMEM((_K,), jnp.int32),
            pltpu.VMEM((_K,), jnp.int32),
            pltpu.VMEM((_K,), jnp.int32),
            pltpu.VMEM((_K, d), jnp.float32),
            pltpu.VMEM((_K, d), jnp.float32),
            pltpu.VMEM_SHARED((n + 8, d), jnp.float32),
            pltpu.SemaphoreType.DMA,
            pltpu.SemaphoreType.DMA,
        ],
    )
    def seg(g_hbm, pk_hbm, out_hbm,
            pflat, sidx_a, sidx_b, didx_c, rows_a, rows_b, acc_sh,
            sem_a, sem_b):
        c = lax.axis_index("c")
        s = lax.axis_index("s")
        wid = s * _NC + c
        pltpu.sync_copy(pk_hbm.at[pl.ds(wid * ewp, ewp)], pflat)

        def fillz(j, carry):
            for t in range(d // 16):
                rows_a[j, pl.ds(t * 16, 16)] = jnp.zeros((16,), jnp.float32)
            return carry

        lax.fori_loop(0, zrow, fillz, 0)

        def zcopy(q, carry):
            pltpu.sync_copy(rows_a.at[pl.ds(0, zrow)],
                            acc_sh.at[pl.ds(s * rpt + q * zrow, zrow)])
            return carry

        lax.fori_loop(0, zq, zcopy, 0)
        plsc.subcore_barrier()

        def gat(i, sidx, buf, sem):
            for t in range(_K // 16):
                v = pflat[pl.ds(i * _K + t * 16, 16)]
                sidx[pl.ds(t * 16, 16)] = v & 0xFFFF
            pltpu.async_copy(g_hbm.at[sidx], buf, sem)

        def scat(i, buf):
            for t in range(_K // 16):
                v = pflat[pl.ds(i * _K + t * 16, 16)]
                didx_c[pl.ds(t * 16, 16)] = lax.shift_right_logical(v, 16)
            pltpu.sync_copy(buf, acc_sh.at[didx_c], add=True)

        gat(0, sidx_a, rows_a, sem_a)

        def body(j, carry):
            i0 = 2 * j
            gat(i0 + 1, sidx_b, rows_b, sem_b)
            pltpu.make_async_copy(g_hbm.at[sidx_a], rows_a, sem_a).wait()
            scat(i0, rows_a)

            @pl.when(j < ncw // 2 - 1)
            def _():
                gat(i0 + 2, sidx_a, rows_a, sem_a)

            pltpu.make_async_copy(g_hbm.at[sidx_b], rows_b, sem_b).wait()
            scat(i0 + 1, rows_b)
            return carry

        lax.fori_loop(0, ncw // 2, body, 0)
        plsc.subcore_barrier()
        pltpu.sync_copy(acc_sh.at[pl.ds(s * rpt, rpt)], out_hbm.at[c, s])

    return seg


def _enc_body(deg_ref, x_ref, w1_ref, g_ref, dinv_ref):
    deg = deg_ref[0, :, 0] + deg_ref[1, :, 0] + 1.0
    dinv = lax.rsqrt(deg)
    g_ref[...] = jnp.dot(x_ref[...] * dinv[:, None], w1_ref[...],
                         preferred_element_type=jnp.float32)
    dinv_ref[...] = dinv[:, None]


def _encode(deg_parts, x, w1, bm):
    n, d = x.shape
    h = w1.shape[1]
    grid = (n // bm,)
    return pl.pallas_call(
        _enc_body,
        grid=grid,
        in_specs=[
            pl.BlockSpec((_NC, bm, 1), lambda i: (0, i, 0)),
            pl.BlockSpec((bm, d), lambda i: (i, 0)),
            pl.BlockSpec((d, h), lambda i: (0, 0)),
        ],
        out_specs=[
            pl.BlockSpec((bm, h), lambda i: (i, 0)),
            pl.BlockSpec((bm, 1), lambda i: (i, 0)),
        ],
        out_shape=[
            jax.ShapeDtypeStruct((n, h), jnp.float32),
            jax.ShapeDtypeStruct((n, 1), jnp.float32),
        ],
    )(deg_parts, x, w1)


def _dec_body(acc_ref, g_ref, dinv_ref, b1_ref, wc_ref, bc_ref, out_ref):
    tot = acc_ref[0] + acc_ref[1] + g_ref[...]
    pre = tot * dinv_ref[...] + b1_ref[...]
    out_ref[...] = jnp.dot(jnp.maximum(pre, 0.0), wc_ref[...],
                           preferred_element_type=jnp.float32) + bc_ref[...]


def _decode(acc_parts, g, dinv, b1, wc, bc, bm):
    n, h = g.shape
    o = wc.shape[1]
    grid = (n // bm,)
    return pl.pallas_call(
        _dec_body,
        grid=grid,
        in_specs=[
            pl.BlockSpec((_NC, bm, h), lambda i: (0, i, 0)),
            pl.BlockSpec((bm, h), lambda i: (i, 0)),
            pl.BlockSpec((bm, 1), lambda i: (i, 0)),
            pl.BlockSpec((1, h), lambda i: (0, 0)),
            pl.BlockSpec((h, o), lambda i: (0, 0)),
            pl.BlockSpec((1, o), lambda i: (0, 0)),
        ],
        out_specs=pl.BlockSpec((bm, o), lambda i: (i, 0)),
        out_shape=jax.ShapeDtypeStruct((n, o), jnp.float32),
    )(acc_parts, g, dinv, b1, wc, bc)


def kernel(x, edge_index, W1, b1, Wc, bc):
    n, d = x.shape
    h = W1.shape[1]
    e = edge_index.shape[1]
    assert n % _NS == 0

    src = edge_index[0].astype(jnp.int32)
    dst = edge_index[1].astype(jnp.int32)

    assert n < 32768
    pk, ewp = _pack_edges(src, dst, n)

    zr = -(-(n + 1) // (_NS * 16)) * 16

    deg_flat = _make_hist(n, ewp, zr)(pk)
    deg_parts = deg_flat.reshape(_NC, 1, _NS * zr)[:, :, :n]
    deg_parts = deg_parts.reshape(_NC, n, 1)

    g, dinv = _encode(deg_parts, x, W1, 1000)

    acc_parts = _make_seg(n, ewp, h)(g, pk).reshape(_NC, n, h)

    out = _decode(acc_parts, g, dinv,
                  b1.reshape(1, h), Wc, bc.reshape(1, -1), 1000)
    return out


# speedup vs baseline: 16.0721x; 1.0397x over previous
import functools

import jax
import jax.numpy as jnp
from jax import lax
from jax.experimental import pallas as pl
from jax.experimental.pallas import tpu as pltpu
from jax.experimental.pallas import tpu_sc as plsc

_NC = 2
_NS = 16
_NW = _NC * _NS

_K = 128


def _mesh():
    return plsc.VectorSubcoreMesh(core_axis_name="c", subcore_axis_name="s")


def _pack_edges(src, dst, n):
    e = src.shape[0]
    ew = -(-e // _NW)
    ewp = -(-ew // (2 * _K)) * (2 * _K)
    packed = src | (dst << 16)
    if e < _NW * ew:
        packed = jnp.concatenate(
            [packed, jnp.full((_NW * ew - e,), jnp.int32(n << 16))])
    pad1 = n + (jnp.arange(ewp - ew, dtype=jnp.int32) % 128)
    pad = jnp.broadcast_to(pad1 << 16, (_NW, ewp - ew))
    out = jnp.concatenate([packed.reshape(_NW, ew), pad], axis=1)
    return out.reshape(-1), ewp


def _make_hist(n, ewp, zr):
    ncw = ewp // _K

    @functools.partial(
        pl.kernel,
        out_type=jax.ShapeDtypeStruct((_NC * _NS * zr,), jnp.float32),
        mesh=_mesh(),
        scratch_types=[
            pltpu.VMEM((ewp,), jnp.int32),
            pltpu.VMEM((_K,), jnp.int32),
            pltpu.VMEM((_K,), jnp.float32),
            pltpu.VMEM((zr,), jnp.float32),
            pltpu.VMEM_SHARED((_NS * zr,), jnp.float32),
        ],
    )
    def hist(pk_hbm, out_hbm, pflat, didx_c, ones, zbuf, flat_sh):
        c = lax.axis_index("c")
        s = lax.axis_index("s")
        wid = s * _NC + c
        pltpu.sync_copy(pk_hbm.at[pl.ds(wid * ewp, ewp)], pflat)

        def fill1(j, carry):
            ones[pl.ds(j * 16, 16)] = jnp.full((16,), 1.0, jnp.float32)
            return carry

        lax.fori_loop(0, _K // 16, fill1, 0)

        def fill0(j, carry):
            zbuf[pl.ds(j * 16, 16)] = jnp.zeros((16,), jnp.float32)
            return carry

        lax.fori_loop(0, zr // 16, fill0, 0)
        pltpu.sync_copy(zbuf, flat_sh.at[pl.ds(s * zr, zr)])
        plsc.subcore_barrier()

        def body(i, carry):
            for t in range(_K // 16):
                v = pflat[pl.ds(i * _K + t * 16, 16)]
                didx_c[pl.ds(t * 16, 16)] = lax.shift_right_logical(v, 16)
            pltpu.sync_copy(ones, flat_sh.at[didx_c], add=True)
            return carry

        lax.fori_loop(0, ncw, body, 0)
        plsc.subcore_barrier()
        pltpu.sync_copy(flat_sh.at[pl.ds(s * zr, zr)],
                        out_hbm.at[pl.ds((c * _NS + s) * zr, zr)])

    return hist


def _make_seg(n, ewp, d):
    rpt = n // _NS
    zq = 5
    zrow = rpt // zq
    ncw = ewp // _K

    @functools.partial(
        pl.kernel,
        out_type=jax.ShapeDtypeStruct((_NC, _NS, rpt, d), jnp.float32),
        mesh=_mesh(),
        scratch_types=[
            pltpu.VMEM((ewp,), jnp.int32),
            pltpu.VMEM((_K,), jnp.int32),
            pltpu.VMEM((_K,), jnp.int32),
            pltpu.VMEM((_K,), jnp.int32),
            pltpu.VMEM((_K, d), jnp.float32),
            pltpu.VMEM((_K, d), jnp.float32),
            pltpu.VMEM_SHARED((n + 128, d), jnp.float32),
            pltpu.SemaphoreType.DMA,
            pltpu.SemaphoreType.DMA,
        ],
    )
    def seg(g_hbm, pk_hbm, out_hbm,
            pflat, sidx_a, sidx_b, didx_c, rows_a, rows_b, acc_sh,
            sem_a, sem_b):
        c = lax.axis_index("c")
        s = lax.axis_index("s")
        wid = s * _NC + c
        pltpu.sync_copy(pk_hbm.at[pl.ds(wid * ewp, ewp)], pflat)

        def fillz(j, carry):
            for t in range(d // 16):
                rows_a[j, pl.ds(t * 16, 16)] = jnp.zeros((16,), jnp.float32)
            return carry

        lax.fori_loop(0, zrow, fillz, 0)

        def zcopy(q, carry):
            pltpu.sync_copy(rows_a.at[pl.ds(0, zrow)],
                            acc_sh.at[pl.ds(s * rpt + q * zrow, zrow)])
            return carry

        lax.fori_loop(0, zq, zcopy, 0)
        plsc.subcore_barrier()

        def gat(i, sidx, buf, sem):
            for t in range(_K // 16):
                v = pflat[pl.ds(i * _K + t * 16, 16)]
                sidx[pl.ds(t * 16, 16)] = v & 0xFFFF
            pltpu.async_copy(g_hbm.at[sidx], buf, sem)

        def scat(i, buf):
            for t in range(_K // 16):
                v = pflat[pl.ds(i * _K + t * 16, 16)]
                didx_c[pl.ds(t * 16, 16)] = lax.shift_right_logical(v, 16)
            pltpu.sync_copy(buf, acc_sh.at[didx_c], add=True)

        gat(0, sidx_a, rows_a, sem_a)

        def body(j, carry):
            i0 = 2 * j
            gat(i0 + 1, sidx_b, rows_b, sem_b)
            pltpu.make_async_copy(g_hbm.at[sidx_a], rows_a, sem_a).wait()
            scat(i0, rows_a)

            @pl.when(j < ncw // 2 - 1)
            def _():
                gat(i0 + 2, sidx_a, rows_a, sem_a)

            pltpu.make_async_copy(g_hbm.at[sidx_b], rows_b, sem_b).wait()
            scat(i0 + 1, rows_b)
            return carry

        lax.fori_loop(0, ncw // 2, body, 0)
        plsc.subcore_barrier()
        pltpu.sync_copy(acc_sh.at[pl.ds(s * rpt, rpt)], out_hbm.at[c, s])

    return seg


def _enc_body(deg_ref, x_ref, w1_ref, g_ref, dinv_ref):
    deg = deg_ref[0, :, 0] + deg_ref[1, :, 0] + 1.0
    dinv = lax.rsqrt(deg)
    g_ref[...] = jnp.dot(x_ref[...] * dinv[:, None], w1_ref[...],
                         preferred_element_type=jnp.float32)
    dinv_ref[...] = dinv[:, None]


def _encode(deg_parts, x, w1, bm):
    n, d = x.shape
    h = w1.shape[1]
    grid = (n // bm,)
    return pl.pallas_call(
        _enc_body,
        grid=grid,
        in_specs=[
            pl.BlockSpec((_NC, bm, 1), lambda i: (0, i, 0)),
            pl.BlockSpec((bm, d), lambda i: (i, 0)),
            pl.BlockSpec((d, h), lambda i: (0, 0)),
        ],
        out_specs=[
            pl.BlockSpec((bm, h), lambda i: (i, 0)),
            pl.BlockSpec((bm, 1), lambda i: (i, 0)),
        ],
        out_shape=[
            jax.ShapeDtypeStruct((n, h), jnp.float32),
            jax.ShapeDtypeStruct((n, 1), jnp.float32),
        ],
    )(deg_parts, x, w1)


def _dec_body(acc_ref, g_ref, dinv_ref, b1_ref, wc_ref, bc_ref, out_ref):
    tot = acc_ref[0] + acc_ref[1] + g_ref[...]
    pre = tot * dinv_ref[...] + b1_ref[...]
    out_ref[...] = jnp.dot(jnp.maximum(pre, 0.0), wc_ref[...],
                           preferred_element_type=jnp.float32) + bc_ref[...]


def _decode(acc_parts, g, dinv, b1, wc, bc, bm):
    n, h = g.shape
    o = wc.shape[1]
    grid = (n // bm,)
    return pl.pallas_call(
        _dec_body,
        grid=grid,
        in_specs=[
            pl.BlockSpec((_NC, bm, h), lambda i: (0, i, 0)),
            pl.BlockSpec((bm, h), lambda i: (i, 0)),
            pl.BlockSpec((bm, 1), lambda i: (i, 0)),
            pl.BlockSpec((1, h), lambda i: (0, 0)),
            pl.BlockSpec((h, o), lambda i: (0, 0)),
            pl.BlockSpec((1, o), lambda i: (0, 0)),
        ],
        out_specs=pl.BlockSpec((bm, o), lambda i: (i, 0)),
        out_shape=jax.ShapeDtypeStruct((n, o), jnp.float32),
    )(acc_parts, g, dinv, b1, wc, bc)


def kernel(x, edge_index, W1, b1, Wc, bc):
    n, d = x.shape
    h = W1.shape[1]
    e = edge_index.shape[1]
    assert n % _NS == 0

    src = edge_index[0].astype(jnp.int32)
    dst = edge_index[1].astype(jnp.int32)

    assert n < 32768
    pk, ewp = _pack_edges(src, dst, n)

    zr = -(-(n + 128) // (_NS * 16)) * 16

    deg_flat = _make_hist(n, ewp, zr)(pk)
    deg_parts = deg_flat.reshape(_NC, 1, _NS * zr)[:, :, :n]
    deg_parts = deg_parts.reshape(_NC, n, 1)

    g, dinv = _encode(deg_parts, x, W1, 1000)

    acc_parts = _make_seg(n, ewp, h)(g, pk).reshape(_NC, n, h)

    out = _decode(acc_parts, g, dinv,
                  b1.reshape(1, h), Wc, bc.reshape(1, -1), 1000)
    return out


# speedup vs baseline: 39.6293x; 2.4657x over previous
import functools

import jax
import jax.numpy as jnp
from jax import lax
from jax.experimental import pallas as pl
from jax.experimental.pallas import tpu as pltpu
from jax.experimental.pallas import tpu_sc as plsc

_NC = 2
_NS = 16
_NW = _NC * _NS

_K = 128


def _mesh():
    return plsc.VectorSubcoreMesh(core_axis_name="c", subcore_axis_name="s")


def _pack_edges(src, dst, n):
    e = src.shape[0]
    ew = -(-e // _NW)
    ewp = -(-ew // (2 * _K)) * (2 * _K)
    packed = src | (dst << 16)
    if e < _NW * ew:
        packed = jnp.concatenate(
            [packed, jnp.full((_NW * ew - e,), jnp.int32(n << 16))])
    pad1 = n + (jnp.arange(ewp - ew, dtype=jnp.int32) % 128)
    pad = jnp.broadcast_to(pad1 << 16, (_NW, ewp - ew))
    out = jnp.concatenate([packed.reshape(_NW, ew), pad], axis=1)
    return out.reshape(-1), ewp


def _make_hist(n, ewp, zr):
    ncw = ewp // _K

    @functools.partial(
        pl.kernel,
        out_type=jax.ShapeDtypeStruct((_NC * _NS * zr,), jnp.float32),
        mesh=_mesh(),
        scratch_types=[
            pltpu.VMEM((ewp,), jnp.int32),
            pltpu.VMEM((_K,), jnp.int32),
            pltpu.VMEM((_K,), jnp.float32),
            pltpu.VMEM((zr,), jnp.float32),
            pltpu.VMEM_SHARED((_NS * zr,), jnp.float32),
        ],
    )
    def hist(pk_hbm, out_hbm, pflat, didx_c, ones, zbuf, flat_sh):
        c = lax.axis_index("c")
        s = lax.axis_index("s")
        wid = s * _NC + c
        pltpu.sync_copy(pk_hbm.at[pl.ds(wid * ewp, ewp)], pflat)

        def fill1(j, carry):
            ones[pl.ds(j * 16, 16)] = jnp.full((16,), 1.0, jnp.float32)
            return carry

        lax.fori_loop(0, _K // 16, fill1, 0)

        def fill0(j, carry):
            zbuf[pl.ds(j * 16, 16)] = jnp.zeros((16,), jnp.float32)
            return carry

        lax.fori_loop(0, zr // 16, fill0, 0)
        pltpu.sync_copy(zbuf, flat_sh.at[pl.ds(s * zr, zr)])
        plsc.subcore_barrier()

        def body(i, carry):
            for t in range(_K // 16):
                v = pflat[pl.ds(i * _K + t * 16, 16)]
                didx_c[pl.ds(t * 16, 16)] = lax.shift_right_logical(v, 16)
            pltpu.sync_copy(ones, flat_sh.at[didx_c], add=True)
            return carry

        lax.fori_loop(0, ncw, body, 0)
        plsc.subcore_barrier()
        pltpu.sync_copy(flat_sh.at[pl.ds(s * zr, zr)],
                        out_hbm.at[pl.ds((c * _NS + s) * zr, zr)])

    return hist


def _make_seg(n, ewp, ew, d):
    rpt = n // _NS
    zq = 5
    zrow = rpt // zq
    full = (ew // _K) & ~1
    rem = ew - full * _K
    assert rem % 16 == 0 and rem <= _K and full >= 2

    @functools.partial(
        pl.kernel,
        out_type=jax.ShapeDtypeStruct((_NC, _NS, rpt, d), jnp.float32),
        mesh=_mesh(),
        scratch_types=[
            pltpu.VMEM((ewp,), jnp.int32),
            pltpu.VMEM((_K,), jnp.int32),
            pltpu.VMEM((_K,), jnp.int32),
            pltpu.VMEM((_K,), jnp.int32),
            pltpu.VMEM((max(rem, 16),), jnp.int32),
            pltpu.VMEM((max(rem, 16),), jnp.int32),
            pltpu.VMEM((_K, d), jnp.float32),
            pltpu.VMEM((_K, d), jnp.float32),
            pltpu.VMEM_SHARED((n, d), jnp.float32),
            pltpu.SemaphoreType.DMA,
            pltpu.SemaphoreType.DMA,
        ],
    )
    def seg(g_hbm, pk_hbm, out_hbm,
            pflat, sidx_a, sidx_b, didx_c, sidx_t, didx_t, rows_a, rows_b,
            acc_sh, sem_a, sem_b):
        c = lax.axis_index("c")
        s = lax.axis_index("s")
        wid = s * _NC + c
        pltpu.sync_copy(pk_hbm.at[pl.ds(wid * ewp, ewp)], pflat)

        def fillz(j, carry):
            for t in range(d // 16):
                rows_a[j, pl.ds(t * 16, 16)] = jnp.zeros((16,), jnp.float32)
            return carry

        lax.fori_loop(0, zrow, fillz, 0)

        def zcopy(q, carry):
            pltpu.sync_copy(rows_a.at[pl.ds(0, zrow)],
                            acc_sh.at[pl.ds(s * rpt + q * zrow, zrow)])
            return carry

        lax.fori_loop(0, zq, zcopy, 0)
        plsc.subcore_barrier()

        def gat(i, sidx, buf, sem):
            for t in range(_K // 16):
                v = pflat[pl.ds(i * _K + t * 16, 16)]
                sidx[pl.ds(t * 16, 16)] = v & 0xFFFF
            pltpu.async_copy(g_hbm.at[sidx], buf, sem)

        def scat(i, buf):
            for t in range(_K // 16):
                v = pflat[pl.ds(i * _K + t * 16, 16)]
                didx_c[pl.ds(t * 16, 16)] = lax.shift_right_logical(v, 16)
            pltpu.sync_copy(buf, acc_sh.at[didx_c], add=True)

        gat(0, sidx_a, rows_a, sem_a)

        def body(j, carry):
            i0 = 2 * j
            gat(i0 + 1, sidx_b, rows_b, sem_b)
            pltpu.make_async_copy(g_hbm.at[sidx_a], rows_a, sem_a).wait()
            scat(i0, rows_a)

            @pl.when(j < full // 2 - 1)
            def _():
                gat(i0 + 2, sidx_a, rows_a, sem_a)

            pltpu.make_async_copy(g_hbm.at[sidx_b], rows_b, sem_b).wait()
            scat(i0 + 1, rows_b)
            return carry

        lax.fori_loop(0, full // 2, body, 0)

        if rem:
            for t in range(rem // 16):
                v = pflat[pl.ds(full * _K + t * 16, 16)]
                sidx_t[pl.ds(t * 16, 16)] = v & 0xFFFF
                didx_t[pl.ds(t * 16, 16)] = lax.shift_right_logical(v, 16)
            pltpu.async_copy(g_hbm.at[sidx_t],
                             rows_a.at[pl.ds(0, rem)], sem_a).wait()
            pltpu.sync_copy(rows_a.at[pl.ds(0, rem)],
                            acc_sh.at[didx_t], add=True)

        plsc.subcore_barrier()
        pltpu.sync_copy(acc_sh.at[pl.ds(s * rpt, rpt)], out_hbm.at[c, s])

    return seg


def _enc_body(deg_ref, x_ref, w1_ref, g_ref, dinv_ref):
    deg = deg_ref[0, :, 0] + deg_ref[1, :, 0] + 1.0
    dinv = lax.rsqrt(deg)
    g_ref[...] = jnp.dot(x_ref[...] * dinv[:, None], w1_ref[...],
                         preferred_element_type=jnp.float32)
    dinv_ref[...] = dinv[:, None]


def _encode(deg_parts, x, w1, bm):
    n, d = x.shape
    h = w1.shape[1]
    grid = (n // bm,)
    return pl.pallas_call(
        _enc_body,
        grid=grid,
        in_specs=[
            pl.BlockSpec((_NC, bm, 1), lambda i: (0, i, 0)),
            pl.BlockSpec((bm, d), lambda i: (i, 0)),
            pl.BlockSpec((d, h), lambda i: (0, 0)),
        ],
        out_specs=[
            pl.BlockSpec((bm, h), lambda i: (i, 0)),
            pl.BlockSpec((bm, 1), lambda i: (i, 0)),
        ],
        out_shape=[
            jax.ShapeDtypeStruct((n, h), jnp.float32),
            jax.ShapeDtypeStruct((n, 1), jnp.float32),
        ],
    )(deg_parts, x, w1)


def _dec_body(acc_ref, g_ref, dinv_ref, b1_ref, wc_ref, bc_ref, out_ref):
    tot = acc_ref[0] + acc_ref[1] + g_ref[...]
    pre = tot * dinv_ref[...] + b1_ref[...]
    out_ref[...] = jnp.dot(jnp.maximum(pre, 0.0), wc_ref[...],
                           preferred_element_type=jnp.float32) + bc_ref[...]


def _decode(acc_parts, g, dinv, b1, wc, bc, bm):
    n, h = g.shape
    o = wc.shape[1]
    grid = (n // bm,)
    return pl.pallas_call(
        _dec_body,
        grid=grid,
        in_specs=[
            pl.BlockSpec((_NC, bm, h), lambda i: (0, i, 0)),
            pl.BlockSpec((bm, h), lambda i: (i, 0)),
            pl.BlockSpec((bm, 1), lambda i: (i, 0)),
            pl.BlockSpec((1, h), lambda i: (0, 0)),
            pl.BlockSpec((h, o), lambda i: (0, 0)),
            pl.BlockSpec((1, o), lambda i: (0, 0)),
        ],
        out_specs=pl.BlockSpec((bm, o), lambda i: (i, 0)),
        out_shape=jax.ShapeDtypeStruct((n, o), jnp.float32),
    )(acc_parts, g, dinv, b1, wc, bc)


def kernel(x, edge_index, W1, b1, Wc, bc):
    n, d = x.shape
    h = W1.shape[1]
    e = edge_index.shape[1]
    assert n % _NS == 0

    src = edge_index[0].astype(jnp.int32)
    dst = edge_index[1].astype(jnp.int32)

    assert n < 32768 and e % _NW == 0
    pk, ewp = _pack_edges(src, dst, n)

    zr = -(-(n + 128) // (_NS * 16)) * 16

    deg_flat = _make_hist(n, ewp, zr)(pk)
    deg_parts = deg_flat.reshape(_NC, 1, _NS * zr)[:, :, :n]
    deg_parts = deg_parts.reshape(_NC, n, 1)

    g, dinv = _encode(deg_parts, x, W1, 1000)

    ew = -(-e // _NW)
    acc_parts = _make_seg(n, ewp, ew, h)(g, pk).reshape(_NC, n, h)

    out = _decode(acc_parts, g, dinv,
                  b1.reshape(1, h), Wc, bc.reshape(1, -1), 1000)
    return out
